# baseline (speedup 1.0000x reference)
import functools

import jax
import jax.numpy as jnp
from jax import lax
from jax.experimental import pallas as pl
from jax.experimental.pallas import tpu as pltpu
from jax.experimental.pallas import tpu_sc as plsc

B, T, D, H = 8, 2048, 512, 512
S = T
TB = 256
NT = T // TB

NW = 32
GS = 128
NG = S // GS
CHUNK = 64


def _seg_body(x_ref, u_ref, m_ref, W1_ref, b1_ref, W2_ref, b2_ref,
              seg_ref, peritem_ref, eye_ref, tri_ref, carry_ref):
    b = pl.program_id(0)
    t = pl.program_id(1)

    @pl.when(jnp.logical_and(b == 0, t == 0))
    def _build():
        ri = jax.lax.broadcasted_iota(jnp.int32, (TB, TB), 0)
        ci = jax.lax.broadcasted_iota(jnp.int32, (TB, TB), 1)
        eye_ref[...] = (ri == ci).astype(jnp.bfloat16)
        tri_ref[...] = (ri <= ci).astype(jnp.bfloat16)

    @pl.when(t == 0)
    def _reset():
        carry_ref[0, 0] = 0.0

    x = x_ref[0]
    h = jnp.maximum(
        jnp.dot(x, W1_ref[...], preferred_element_type=jnp.float32)
        + b1_ref[...], 0.0)
    logits = (jnp.dot(h, W2_ref[...], preferred_element_type=jnp.float32)
              + b2_ref[0, 0])
    u = u_ref[0, 0]
    noise = jnp.log(u) - jnp.log1p(-u)
    soft = jax.nn.sigmoid(logits + noise)
    hard = (soft > 0.5).astype(jnp.float32) * m_ref[0, 0]

    hard_row = jax.lax.dot_general(
        hard.astype(jnp.bfloat16), eye_ref[...], (((0,), (0,)), ((), ())),
        preferred_element_type=jnp.float32)
    cs_row = jnp.dot(hard_row.astype(jnp.bfloat16), tri_ref[...],
                     preferred_element_type=jnp.float32)
    carry = carry_ref[0, 0]
    seg_row = carry + cs_row - hard_row
    seg_ref[...] = seg_row.astype(jnp.int32).reshape(1, 1, 1, TB)
    total = carry + jnp.sum(hard)
    carry_ref[0, 0] = total
    peritem_ref[...] = jnp.full((1, 1, 128), total, dtype=jnp.float32)


def _make_sc_pool():
    mesh = plsc.VectorSubcoreMesh(core_axis_name="c", subcore_axis_name="s")

    @functools.partial(
        pl.kernel, mesh=mesh,
        out_type=jax.ShapeDtypeStruct((B * S, D), jnp.float32),
        scratch_types=[
            pltpu.VMEM((T + 16,), jnp.int32),
            pltpu.VMEM((CHUNK, D), jnp.float32),
            pltpu.VMEM((GS, D), jnp.float32),
            pltpu.VMEM((GS, 16), jnp.float32),
        ],
    )
    def sc_pool(hid_hbm, seg_hbm, out_hbm, segs_v, rowbuf, accum, cnt):
        wid = lax.axis_index("s") * 2 + lax.axis_index("c")

        def sld(i):
            return segs_v[pl.ds(i, 16)][0]

        def lower_bound(target):
            def bs(_, lo_hi):
                lo, hi = lo_hi
                mid = (lo + hi) // 2
                sm = sld(mid)
                lo2 = jnp.where(sm < target, mid + 1, lo)
                hi2 = jnp.where(sm < target, hi, mid)
                return lo2, hi2
            lo, _ = lax.fori_loop(0, 11, bs, (jnp.int32(0), jnp.int32(T)))
            return lo

        def vtask(i, _):
            v = wid + NW * i
            b = v % B
            g = v // B
            seg_base = g * GS
            pltpu.sync_copy(seg_hbm.at[pl.ds(b * T, T)],
                            segs_v.at[pl.ds(0, T)])
            t0 = lower_bound(seg_base)
            t1 = lower_bound(seg_base + GS)
            maxseg = sld(jnp.int32(T - 1))
            used = jnp.clip(maxseg + 1 - seg_base, 0, GS)

            ones16 = jnp.ones((16,), jnp.float32)

            a0 = (t0 // 8) * 8

            def chunk(k, prev):
                ak = a0 + CHUNK * k
                ck = jnp.minimum(ak, T - CHUNK)
                pltpu.sync_copy(hid_hbm.at[pl.ds(b * T + ck, CHUNK)], rowbuf)
                begin = jnp.maximum(ak, t0)
                end = jnp.minimum(ak + CHUNK, t1)

                def tok(tt, prev):
                    sg = sld(tt)
                    r = sg - seg_base
                    row = tt - ck
                    first = sg != prev

                    @pl.when(first)
                    def _store():
                        for j in range(D // 16):
                            accum[r, pl.ds(16 * j, 16)] = (
                                rowbuf[row, pl.ds(16 * j, 16)])
                        cnt[r, :] = ones16

                    @pl.when(jnp.logical_not(first))
                    def _add():
                        for j in range(D // 16):
                            plsc.addupdate(accum.at[r, pl.ds(16 * j, 16)],
                                           rowbuf[row, pl.ds(16 * j, 16)])
                        plsc.addupdate(cnt.at[r], ones16)

                    return sg

                return lax.fori_loop(begin, end, tok, prev)

            nk = (t1 - a0 + CHUNK - 1) // CHUNK
            lax.fori_loop(0, nk, chunk, jnp.int32(-1))

            def fin(r, _):
                @pl.when(r < used)
                def _scale():
                    inv = 1.0 / (cnt[r, :] + 1e-9)
                    for j in range(D // 16):
                        accum[r, pl.ds(16 * j, 16)] = (
                            accum[r, pl.ds(16 * j, 16)] * inv)

                @pl.when(r >= used)
                def _zero():
                    for j in range(D // 16):
                        accum[r, pl.ds(16 * j, 16)] = jnp.zeros(
                            (16,), jnp.float32)

                return 0

            lax.fori_loop(0, GS, fin, 0)
            pltpu.sync_copy(accum, out_hbm.at[pl.ds(b * S + seg_base, GS)])
            return 0

        lax.fori_loop(0, (B * NG) // NW, vtask, 0)

    return sc_pool


_sc_pool = _make_sc_pool()


def _lgamma(x):
    z = x + 8.0
    zi = 1.0 / z
    zi2 = zi * zi
    series = ((z - 0.5) * jnp.log(z) - z + 0.9189385332046727
              + zi * (1.0 / 12.0 + zi2 * (-1.0 / 360.0 + zi2 / 1260.0)))
    prod = (x * (x + 1.0) * (x + 2.0) * (x + 3.0)
            * (x + 4.0) * (x + 5.0) * (x + 6.0) * (x + 7.0))
    return series - jnp.log(prod)


def _loss_body(nb_ref, tgt_ref, m_ref,
               loss_ref, numb_ref, totpos_ref, short_ref):
    nb = nb_ref[...]
    tgt = tgt_ref[...]
    m = m_ref[...]
    totals = jnp.sum(m, axis=1, keepdims=True)
    p = jnp.clip(tgt / totals, 1e-6, 1.0 - 1e-6)
    log_prob = (_lgamma(totals + 1.0) - _lgamma(nb + 1.0)
                - _lgamma(totals - nb + 1.0)
                + nb * jnp.log(p) + (totals - nb) * jnp.log1p(-p))
    loss = -log_prob / totals
    loss_ref[...] = jnp.full((1, 1), jnp.sum(loss) / B, dtype=jnp.float32)
    numb_ref[...] = jnp.full((1, 1), jnp.sum(nb), dtype=jnp.float32)
    totpos_ref[...] = jnp.full((1, 1), jnp.sum(m), dtype=jnp.float32)
    s_iota = jax.lax.broadcasted_iota(jnp.int32, (B, S), 1).astype(jnp.float32)
    short_ref[...] = (s_iota < nb).astype(jnp.float32)


def kernel(hidden, attention_mask, target_boundary_counts,
           W1, b1, W2, b2, gumbel_u):
    gu = gumbel_u.reshape(B, NT, TB, 1)
    mk = attention_mask.reshape(B, NT, TB, 1)
    b1r = b1.reshape(1, H)
    b2r = jnp.broadcast_to(b2.reshape(1, 1), (1, 128))

    seg_ids, peritem = pl.pallas_call(
        _seg_body,
        grid=(B, NT),
        in_specs=[
            pl.BlockSpec((1, TB, D), lambda b, t: (b, t, 0)),
            pl.BlockSpec((1, 1, TB, 1), lambda b, t: (b, t, 0, 0)),
            pl.BlockSpec((1, 1, TB, 1), lambda b, t: (b, t, 0, 0)),
            pl.BlockSpec((D, H), lambda b, t: (0, 0)),
            pl.BlockSpec((1, H), lambda b, t: (0, 0)),
            pl.BlockSpec((H, 1), lambda b, t: (0, 0)),
            pl.BlockSpec((1, 128), lambda b, t: (0, 0)),
        ],
        out_specs=[
            pl.BlockSpec((1, 1, 1, TB), lambda b, t: (b, t, 0, 0)),
            pl.BlockSpec((1, 1, 128), lambda b, t: (b, 0, 0)),
        ],
        out_shape=[
            jax.ShapeDtypeStruct((B, NT, 1, TB), jnp.int32),
            jax.ShapeDtypeStruct((B, 1, 128), jnp.float32),
        ],
        scratch_shapes=[
            pltpu.VMEM((TB, TB), jnp.bfloat16),
            pltpu.VMEM((TB, TB), jnp.bfloat16),
            pltpu.SMEM((1, 1), jnp.float32),
        ],
        compiler_params=pltpu.CompilerParams(
            dimension_semantics=("arbitrary", "arbitrary")),
    )(hidden, gu, mk, W1, b1r, W2, b2r)

    pooled = _sc_pool(hidden.reshape(B * T, D),
                      seg_ids.reshape(B * T)).reshape(B, S, D)

    nb_col = peritem[:, 0, :1]
    tgt_col = target_boundary_counts.astype(jnp.float32).reshape(B, 1)

    loss2, numb2, totpos2, shortened = pl.pallas_call(
        _loss_body,
        out_shape=[
            jax.ShapeDtypeStruct((1, 1), jnp.float32),
            jax.ShapeDtypeStruct((1, 1), jnp.float32),
            jax.ShapeDtypeStruct((1, 1), jnp.float32),
            jax.ShapeDtypeStruct((B, S), jnp.float32),
        ],
    )(nb_col, tgt_col, attention_mask)

    return (pooled, loss2[0, 0], numb2[0, 0], totpos2[0, 0], shortened)


# speedup vs baseline: 1.1088x; 1.1088x over previous
import functools

import jax
import jax.numpy as jnp
from jax import lax
from jax.experimental import pallas as pl
from jax.experimental.pallas import tpu as pltpu
from jax.experimental.pallas import tpu_sc as plsc

B, T, D, H = 8, 2048, 512, 512
S = T
TB = 256
NT = T // TB

NW = 32
GS = 128
NG = S // GS
CHUNK = 32


def _seg_body(x_ref, u_ref, m_ref, W1_ref, b1_ref, W2_ref, b2_ref,
              seg_ref, peritem_ref, eye_ref, tri_ref, carry_ref):
    b = pl.program_id(0)
    t = pl.program_id(1)

    @pl.when(jnp.logical_and(b == 0, t == 0))
    def _build():
        ri = jax.lax.broadcasted_iota(jnp.int32, (TB, TB), 0)
        ci = jax.lax.broadcasted_iota(jnp.int32, (TB, TB), 1)
        eye_ref[...] = (ri == ci).astype(jnp.bfloat16)
        tri_ref[...] = (ri <= ci).astype(jnp.bfloat16)

    @pl.when(t == 0)
    def _reset():
        carry_ref[0, 0] = 0.0

    x = x_ref[0]
    h = jnp.maximum(
        jnp.dot(x, W1_ref[...], preferred_element_type=jnp.float32)
        + b1_ref[...], 0.0)
    logits = (jnp.dot(h, W2_ref[...], preferred_element_type=jnp.float32)
              + b2_ref[0, 0])
    u = u_ref[0, 0]
    noise = jnp.log(u) - jnp.log1p(-u)
    soft = jax.nn.sigmoid(logits + noise)
    hard = (soft > 0.5).astype(jnp.float32) * m_ref[0, 0]

    hard_row = jax.lax.dot_general(
        hard.astype(jnp.bfloat16), eye_ref[...], (((0,), (0,)), ((), ())),
        preferred_element_type=jnp.float32)
    cs_row = jnp.dot(hard_row.astype(jnp.bfloat16), tri_ref[...],
                     preferred_element_type=jnp.float32)
    carry = carry_ref[0, 0]
    seg_row = carry + cs_row - hard_row
    seg_ref[...] = seg_row.astype(jnp.int32).reshape(1, 1, 1, TB)
    total = carry + jnp.sum(hard)
    carry_ref[0, 0] = total
    peritem_ref[...] = jnp.full((1, 1, 128), total, dtype=jnp.float32)


def _make_sc_pool():
    mesh = plsc.VectorSubcoreMesh(core_axis_name="c", subcore_axis_name="s")

    @functools.partial(
        pl.kernel, mesh=mesh,
        out_type=jax.ShapeDtypeStruct((B * S, D), jnp.float32),
        scratch_types=[
            pltpu.VMEM((T + 16,), jnp.int32),
            pltpu.VMEM((2, CHUNK, D), jnp.float32),
            pltpu.VMEM((GS, D), jnp.float32),
            pltpu.VMEM((GS, 16), jnp.float32),
            pltpu.SemaphoreType.DMA,
            pltpu.SemaphoreType.DMA,
        ],
    )
    def sc_pool(hid_hbm, seg_hbm, out_hbm, segs_v, rowbuf, accum, cnt,
                sem0, sem1):
        wid = lax.axis_index("s") * 2 + lax.axis_index("c")

        def sld(i):
            return segs_v[pl.ds(i, 16)][0]

        def lower_bound(target):
            def bs(_, lo_hi):
                lo, hi = lo_hi
                mid = (lo + hi) // 2
                sm = sld(mid)
                lo2 = jnp.where(sm < target, mid + 1, lo)
                hi2 = jnp.where(sm < target, hi, mid)
                return lo2, hi2
            lo, _ = lax.fori_loop(0, 11, bs, (jnp.int32(0), jnp.int32(T)))
            return lo

        def vtask(i, _):
            v = wid + NW * i
            b = v % B
            g = v // B
            seg_base = g * GS
            pltpu.sync_copy(seg_hbm.at[pl.ds(b * T, T)],
                            segs_v.at[pl.ds(0, T)])
            t0 = lower_bound(seg_base)
            t1 = lower_bound(seg_base + GS)
            maxseg = sld(jnp.int32(T - 1))
            used = jnp.clip(maxseg + 1 - seg_base, 0, GS)

            ones16 = jnp.ones((16,), jnp.float32)

            a0 = (t0 // 8) * 8
            nk = (t1 - a0 + CHUNK - 1) // CHUNK

            def issue(k, slot, sem):
                ak = a0 + CHUNK * k
                ck = jnp.minimum(ak, T - CHUNK)
                pltpu.make_async_copy(
                    hid_hbm.at[pl.ds(b * T + ck, CHUNK)],
                    rowbuf.at[slot], sem).start()

            def drain(slot, sem):
                pltpu.make_async_copy(
                    hid_hbm.at[pl.ds(b * T, CHUNK)],
                    rowbuf.at[slot], sem).wait()

            @pl.when(nk > 0)
            def _prime():
                issue(0, 0, sem0)

            def chunk(k, prev):
                ak = a0 + CHUNK * k
                ck = jnp.minimum(ak, T - CHUNK)
                even = (k % 2) == 0

                @pl.when(jnp.logical_and(k + 1 < nk, even))
                def _n1():
                    issue(k + 1, 1, sem1)

                @pl.when(jnp.logical_and(k + 1 < nk,
                                         jnp.logical_not(even)))
                def _n0():
                    issue(k + 1, 0, sem0)

                @pl.when(even)
                def _w0():
                    drain(0, sem0)

                @pl.when(jnp.logical_not(even))
                def _w1():
                    drain(1, sem1)

                slot = k % 2
                begin = jnp.maximum(ak, t0)
                end = jnp.minimum(ak + CHUNK, t1)

                def tok(tt, prev):
                    sg = sld(tt)
                    r = sg - seg_base
                    row = tt - ck
                    first = sg != prev

                    @pl.when(first)
                    def _store():
                        for j in range(D // 16):
                            accum[r, pl.ds(16 * j, 16)] = (
                                rowbuf[slot, row, pl.ds(16 * j, 16)])
                        cnt[r, :] = ones16

                    @pl.when(jnp.logical_not(first))
                    def _add():
                        for j in range(D // 16):
                            plsc.addupdate(accum.at[r, pl.ds(16 * j, 16)],
                                           rowbuf[slot, row, pl.ds(16 * j, 16)])
                        plsc.addupdate(cnt.at[r], ones16)

                    return sg

                return lax.fori_loop(begin, end, tok, prev)

            lax.fori_loop(0, nk, chunk, jnp.int32(-1))

            def scale(r, _):
                inv = 1.0 / (cnt[r, :] + 1e-9)
                for j in range(D // 16):
                    accum[r, pl.ds(16 * j, 16)] = (
                        accum[r, pl.ds(16 * j, 16)] * inv)
                return 0

            def zero(r, _):
                for j in range(D // 16):
                    accum[r, pl.ds(16 * j, 16)] = jnp.zeros(
                        (16,), jnp.float32)
                return 0

            lax.fori_loop(0, used, scale, 0)
            lax.fori_loop(used, GS, zero, 0)
            pltpu.sync_copy(accum, out_hbm.at[pl.ds(b * S + seg_base, GS)])
            return 0

        lax.fori_loop(0, (B * NG) // NW, vtask, 0)

    return sc_pool


_sc_pool = _make_sc_pool()


def _lgamma(x):
    z = x + 8.0
    zi = 1.0 / z
    zi2 = zi * zi
    series = ((z - 0.5) * jnp.log(z) - z + 0.9189385332046727
              + zi * (1.0 / 12.0 + zi2 * (-1.0 / 360.0 + zi2 / 1260.0)))
    prod = (x * (x + 1.0) * (x + 2.0) * (x + 3.0)
            * (x + 4.0) * (x + 5.0) * (x + 6.0) * (x + 7.0))
    return series - jnp.log(prod)


def _loss_body(nb_ref, tgt_ref, m_ref,
               loss_ref, numb_ref, totpos_ref, short_ref):
    nb = nb_ref[...]
    tgt = tgt_ref[...]
    m = m_ref[...]
    totals = jnp.sum(m, axis=1, keepdims=True)
    p = jnp.clip(tgt / totals, 1e-6, 1.0 - 1e-6)
    log_prob = (_lgamma(totals + 1.0) - _lgamma(nb + 1.0)
                - _lgamma(totals - nb + 1.0)
                + nb * jnp.log(p) + (totals - nb) * jnp.log1p(-p))
    loss = -log_prob / totals
    loss_ref[...] = jnp.full((1, 1), jnp.sum(loss) / B, dtype=jnp.float32)
    numb_ref[...] = jnp.full((1, 1), jnp.sum(nb), dtype=jnp.float32)
    totpos_ref[...] = jnp.full((1, 1), jnp.sum(m), dtype=jnp.float32)
    s_iota = jax.lax.broadcasted_iota(jnp.int32, (B, S), 1).astype(jnp.float32)
    short_ref[...] = (s_iota < nb).astype(jnp.float32)


def kernel(hidden, attention_mask, target_boundary_counts,
           W1, b1, W2, b2, gumbel_u):
    gu = gumbel_u.reshape(B, NT, TB, 1)
    mk = attention_mask.reshape(B, NT, TB, 1)
    b1r = b1.reshape(1, H)
    b2r = jnp.broadcast_to(b2.reshape(1, 1), (1, 128))

    seg_ids, peritem = pl.pallas_call(
        _seg_body,
        grid=(B, NT),
        in_specs=[
            pl.BlockSpec((1, TB, D), lambda b, t: (b, t, 0)),
            pl.BlockSpec((1, 1, TB, 1), lambda b, t: (b, t, 0, 0)),
            pl.BlockSpec((1, 1, TB, 1), lambda b, t: (b, t, 0, 0)),
            pl.BlockSpec((D, H), lambda b, t: (0, 0)),
            pl.BlockSpec((1, H), lambda b, t: (0, 0)),
            pl.BlockSpec((H, 1), lambda b, t: (0, 0)),
            pl.BlockSpec((1, 128), lambda b, t: (0, 0)),
        ],
        out_specs=[
            pl.BlockSpec((1, 1, 1, TB), lambda b, t: (b, t, 0, 0)),
            pl.BlockSpec((1, 1, 128), lambda b, t: (b, 0, 0)),
        ],
        out_shape=[
            jax.ShapeDtypeStruct((B, NT, 1, TB), jnp.int32),
            jax.ShapeDtypeStruct((B, 1, 128), jnp.float32),
        ],
        scratch_shapes=[
            pltpu.VMEM((TB, TB), jnp.bfloat16),
            pltpu.VMEM((TB, TB), jnp.bfloat16),
            pltpu.SMEM((1, 1), jnp.float32),
        ],
        compiler_params=pltpu.CompilerParams(
            dimension_semantics=("arbitrary", "arbitrary")),
    )(hidden, gu, mk, W1, b1r, W2, b2r)

    pooled = _sc_pool(hidden.reshape(B * T, D),
                      seg_ids.reshape(B * T)).reshape(B, S, D)

    nb_col = peritem[:, 0, :1]
    tgt_col = target_boundary_counts.astype(jnp.float32).reshape(B, 1)

    loss2, numb2, totpos2, shortened = pl.pallas_call(
        _loss_body,
        out_shape=[
            jax.ShapeDtypeStruct((1, 1), jnp.float32),
            jax.ShapeDtypeStruct((1, 1), jnp.float32),
            jax.ShapeDtypeStruct((1, 1), jnp.float32),
            jax.ShapeDtypeStruct((B, S), jnp.float32),
        ],
    )(nb_col, tgt_col, attention_mask)

    return (pooled, loss2[0, 0], numb2[0, 0], totpos2[0, 0], shortened)


# speedup vs baseline: 1.1364x; 1.0249x over previous
import functools

import jax
import jax.numpy as jnp
from jax import lax
from jax.experimental import pallas as pl
from jax.experimental.pallas import tpu as pltpu
from jax.experimental.pallas import tpu_sc as plsc

B, T, D, H = 8, 2048, 512, 512
S = T
TB = 256
NT = T // TB

NW = 32
GS = 128
NG = S // GS
CHUNK = 32


def _seg_body(x_ref, u_ref, m_ref, W1_ref, b1_ref, W2_ref, b2_ref,
              seg_ref, peritem_ref, eye_ref, tri_ref, carry_ref):
    b = pl.program_id(0)
    t = pl.program_id(1)

    @pl.when(jnp.logical_and(b == 0, t == 0))
    def _build():
        ri = jax.lax.broadcasted_iota(jnp.int32, (TB, TB), 0)
        ci = jax.lax.broadcasted_iota(jnp.int32, (TB, TB), 1)
        eye_ref[...] = (ri == ci).astype(jnp.bfloat16)
        tri_ref[...] = (ri <= ci).astype(jnp.bfloat16)

    @pl.when(t == 0)
    def _reset():
        carry_ref[0, 0] = 0.0

    x = x_ref[0]
    h = jnp.maximum(
        jnp.dot(x, W1_ref[...], preferred_element_type=jnp.float32)
        + b1_ref[...], 0.0)
    logits = (jnp.dot(h, W2_ref[...], preferred_element_type=jnp.float32)
              + b2_ref[0, 0])
    u = u_ref[0, 0]
    noise = jnp.log(u) - jnp.log1p(-u)
    soft = jax.nn.sigmoid(logits + noise)
    hard = (soft > 0.5).astype(jnp.float32) * m_ref[0, 0]

    hard_row = jax.lax.dot_general(
        hard.astype(jnp.bfloat16), eye_ref[...], (((0,), (0,)), ((), ())),
        preferred_element_type=jnp.float32)
    cs_row = jnp.dot(hard_row.astype(jnp.bfloat16), tri_ref[...],
                     preferred_element_type=jnp.float32)
    carry = carry_ref[0, 0]
    seg_row = carry + cs_row - hard_row
    seg_ref[...] = seg_row.astype(jnp.int32).reshape(1, 1, 1, TB)
    total = carry + jnp.sum(hard)
    carry_ref[0, 0] = total
    peritem_ref[...] = jnp.full((1, 1, 128), total, dtype=jnp.float32)


def _make_sc_pool():
    mesh = plsc.VectorSubcoreMesh(core_axis_name="c", subcore_axis_name="s")

    @functools.partial(
        pl.kernel, mesh=mesh,
        out_type=jax.ShapeDtypeStruct((B * S, D), jnp.float32),
        scratch_types=[
            pltpu.VMEM((T + 16,), jnp.int32),
            pltpu.VMEM((2, CHUNK, D), jnp.float32),
            pltpu.VMEM((GS, D), jnp.float32),
            pltpu.VMEM((GS, 16), jnp.float32),
            pltpu.SemaphoreType.DMA,
            pltpu.SemaphoreType.DMA,
        ],
    )
    def sc_pool(hid_hbm, seg_hbm, out_hbm, segs_v, rowbuf, accum, cnt,
                sem0, sem1):
        wid = lax.axis_index("s") * 2 + lax.axis_index("c")
        b = wid % B

        def sld(i):
            return segs_v[pl.ds(i, 16)][0]

        def lower_bound(target):
            def bs(_, lo_hi):
                lo, hi = lo_hi
                mid = (lo + hi) // 2
                sm = sld(mid)
                lo2 = jnp.where(sm < target, mid + 1, lo)
                hi2 = jnp.where(sm < target, hi, mid)
                return lo2, hi2
            lo, _ = lax.fori_loop(0, 11, bs, (jnp.int32(0), jnp.int32(T)))
            return lo

        pltpu.sync_copy(seg_hbm.at[pl.ds(b * T, T)], segs_v.at[pl.ds(0, T)])
        maxseg = sld(jnp.int32(T - 1))

        ones16 = jnp.ones((16,), jnp.float32)

        def zero_rows(lo, hi):
            def zero(r, _):
                for j in range(D // 16):
                    accum[r, pl.ds(16 * j, 16)] = jnp.zeros(
                        (16,), jnp.float32)
                return 0
            lax.fori_loop(lo, hi, zero, 0)

        def vtask(i, dirty):
            v = wid + NW * i
            g = v // B
            seg_base = g * GS
            used = jnp.clip(maxseg + 1 - seg_base, 0, GS)

            @pl.when(jnp.logical_and(used == 0, dirty > 0))
            def _clean():
                zero_rows(jnp.int32(0), jnp.int32(GS))

            @pl.when(used == 0)
            def _empty():
                pltpu.sync_copy(accum,
                                out_hbm.at[pl.ds(b * S + seg_base, GS)])

            @pl.when(used > 0)
            def _work():
                _run_range(seg_base, used)
            return (used > 0).astype(jnp.int32)

        def _run_range(seg_base, used):
            t0 = lower_bound(seg_base)
            t1 = lower_bound(seg_base + GS)

            a0 = (t0 // 8) * 8
            nk = (t1 - a0 + CHUNK - 1) // CHUNK

            def issue(k, slot, sem):
                ak = a0 + CHUNK * k
                ck = jnp.minimum(ak, T - CHUNK)
                pltpu.make_async_copy(
                    hid_hbm.at[pl.ds(b * T + ck, CHUNK)],
                    rowbuf.at[slot], sem).start()

            def drain(slot, sem):
                pltpu.make_async_copy(
                    hid_hbm.at[pl.ds(b * T, CHUNK)],
                    rowbuf.at[slot], sem).wait()

            @pl.when(nk > 0)
            def _prime():
                issue(0, 0, sem0)

            def chunk(k, prev):
                ak = a0 + CHUNK * k
                ck = jnp.minimum(ak, T - CHUNK)
                even = (k % 2) == 0

                @pl.when(jnp.logical_and(k + 1 < nk, even))
                def _n1():
                    issue(k + 1, 1, sem1)

                @pl.when(jnp.logical_and(k + 1 < nk,
                                         jnp.logical_not(even)))
                def _n0():
                    issue(k + 1, 0, sem0)

                @pl.when(even)
                def _w0():
                    drain(0, sem0)

                @pl.when(jnp.logical_not(even))
                def _w1():
                    drain(1, sem1)

                slot = k % 2
                begin = jnp.maximum(ak, t0)
                end = jnp.minimum(ak + CHUNK, t1)

                def tok(tt, prev):
                    sg = sld(tt)
                    r = sg - seg_base
                    row = tt - ck
                    first = sg != prev

                    @pl.when(first)
                    def _store():
                        for j in range(D // 16):
                            accum[r, pl.ds(16 * j, 16)] = (
                                rowbuf[slot, row, pl.ds(16 * j, 16)])
                        cnt[r, :] = ones16

                    @pl.when(jnp.logical_not(first))
                    def _add():
                        for j in range(D // 16):
                            plsc.addupdate(accum.at[r, pl.ds(16 * j, 16)],
                                           rowbuf[slot, row, pl.ds(16 * j, 16)])
                        plsc.addupdate(cnt.at[r], ones16)

                    return sg

                return lax.fori_loop(begin, end, tok, prev)

            lax.fori_loop(0, nk, chunk, jnp.int32(-1))

            def scale(r, _):
                inv = 1.0 / (cnt[r, :] + 1e-9)
                for j in range(D // 16):
                    accum[r, pl.ds(16 * j, 16)] = (
                        accum[r, pl.ds(16 * j, 16)] * inv)
                return 0

            lax.fori_loop(0, used, scale, 0)
            zero_rows(used, jnp.int32(GS))
            pltpu.sync_copy(accum, out_hbm.at[pl.ds(b * S + seg_base, GS)])
            return 0

        lax.fori_loop(0, (B * NG) // NW, vtask, jnp.int32(1))

    return sc_pool


_sc_pool = _make_sc_pool()


def _lgamma(x):
    z = x + 8.0
    zi = 1.0 / z
    zi2 = zi * zi
    series = ((z - 0.5) * jnp.log(z) - z + 0.9189385332046727
              + zi * (1.0 / 12.0 + zi2 * (-1.0 / 360.0 + zi2 / 1260.0)))
    prod = (x * (x + 1.0) * (x + 2.0) * (x + 3.0)
            * (x + 4.0) * (x + 5.0) * (x + 6.0) * (x + 7.0))
    return series - jnp.log(prod)


def _loss_body(nb_ref, tgt_ref, m_ref,
               loss_ref, numb_ref, totpos_ref, short_ref):
    nb = nb_ref[...]
    tgt = tgt_ref[...]
    m = m_ref[...]
    totals = jnp.sum(m, axis=1, keepdims=True)
    p = jnp.clip(tgt / totals, 1e-6, 1.0 - 1e-6)
    log_prob = (_lgamma(totals + 1.0) - _lgamma(nb + 1.0)
                - _lgamma(totals - nb + 1.0)
                + nb * jnp.log(p) + (totals - nb) * jnp.log1p(-p))
    loss = -log_prob / totals
    loss_ref[...] = jnp.full((1, 1), jnp.sum(loss) / B, dtype=jnp.float32)
    numb_ref[...] = jnp.full((1, 1), jnp.sum(nb), dtype=jnp.float32)
    totpos_ref[...] = jnp.full((1, 1), jnp.sum(m), dtype=jnp.float32)
    s_iota = jax.lax.broadcasted_iota(jnp.int32, (B, S), 1).astype(jnp.float32)
    short_ref[...] = (s_iota < nb).astype(jnp.float32)


def kernel(hidden, attention_mask, target_boundary_counts,
           W1, b1, W2, b2, gumbel_u):
    gu = gumbel_u.reshape(B, NT, TB, 1)
    mk = attention_mask.reshape(B, NT, TB, 1)
    b1r = b1.reshape(1, H)
    b2r = jnp.broadcast_to(b2.reshape(1, 1), (1, 128))

    seg_ids, peritem = pl.pallas_call(
        _seg_body,
        grid=(B, NT),
        in_specs=[
            pl.BlockSpec((1, TB, D), lambda b, t: (b, t, 0)),
            pl.BlockSpec((1, 1, TB, 1), lambda b, t: (b, t, 0, 0)),
            pl.BlockSpec((1, 1, TB, 1), lambda b, t: (b, t, 0, 0)),
            pl.BlockSpec((D, H), lambda b, t: (0, 0)),
            pl.BlockSpec((1, H), lambda b, t: (0, 0)),
            pl.BlockSpec((H, 1), lambda b, t: (0, 0)),
            pl.BlockSpec((1, 128), lambda b, t: (0, 0)),
        ],
        out_specs=[
            pl.BlockSpec((1, 1, 1, TB), lambda b, t: (b, t, 0, 0)),
            pl.BlockSpec((1, 1, 128), lambda b, t: (b, 0, 0)),
        ],
        out_shape=[
            jax.ShapeDtypeStruct((B, NT, 1, TB), jnp.int32),
            jax.ShapeDtypeStruct((B, 1, 128), jnp.float32),
        ],
        scratch_shapes=[
            pltpu.VMEM((TB, TB), jnp.bfloat16),
            pltpu.VMEM((TB, TB), jnp.bfloat16),
            pltpu.SMEM((1, 1), jnp.float32),
        ],
        compiler_params=pltpu.CompilerParams(
            dimension_semantics=("arbitrary", "arbitrary")),
    )(hidden, gu, mk, W1, b1r, W2, b2r)

    pooled = _sc_pool(hidden.reshape(B * T, D),
                      seg_ids.reshape(B * T)).reshape(B, S, D)

    nb_col = peritem[:, 0, :1]
    tgt_col = target_boundary_counts.astype(jnp.float32).reshape(B, 1)

    loss2, numb2, totpos2, shortened = pl.pallas_call(
        _loss_body,
        out_shape=[
            jax.ShapeDtypeStruct((1, 1), jnp.float32),
            jax.ShapeDtypeStruct((1, 1), jnp.float32),
            jax.ShapeDtypeStruct((1, 1), jnp.float32),
            jax.ShapeDtypeStruct((B, S), jnp.float32),
        ],
    )(nb_col, tgt_col, attention_mask)

    return (pooled, loss2[0, 0], numb2[0, 0], totpos2[0, 0], shortened)


# speedup vs baseline: 1.2713x; 1.1188x over previous
import functools

import jax
import jax.numpy as jnp
from jax import lax
from jax.experimental import pallas as pl
from jax.experimental.pallas import tpu as pltpu
from jax.experimental.pallas import tpu_sc as plsc

B, T, D, H = 8, 2048, 512, 512
S = T
TB = 512
NT = T // TB

NW = 32
GS = 128
NG = S // GS
CHUNK = 32


def _seg_body(x_ref, u_ref, m_ref, W1_ref, b1_ref, W2_ref, b2_ref,
              seg_ref, peritem_ref, eye_ref, tri_ref, carry_ref):
    b = pl.program_id(0)
    t = pl.program_id(1)

    @pl.when(jnp.logical_and(b == 0, t == 0))
    def _build():
        ri = jax.lax.broadcasted_iota(jnp.int32, (TB, TB), 0)
        ci = jax.lax.broadcasted_iota(jnp.int32, (TB, TB), 1)
        eye_ref[...] = (ri == ci).astype(jnp.bfloat16)
        tri_ref[...] = (ri <= ci).astype(jnp.bfloat16)

    @pl.when(t == 0)
    def _reset():
        carry_ref[0, 0] = 0.0

    x = x_ref[0]
    h = jnp.maximum(
        jnp.dot(x, W1_ref[...], preferred_element_type=jnp.float32)
        + b1_ref[...], 0.0)
    logits = (jnp.dot(h, W2_ref[...], preferred_element_type=jnp.float32)
              + b2_ref[0, 0])
    u = u_ref[0, 0]
    noise = jnp.log(u) - jnp.log1p(-u)
    soft = jax.nn.sigmoid(logits + noise)
    hard = (soft > 0.5).astype(jnp.float32) * m_ref[0, 0]

    hard_row = jax.lax.dot_general(
        hard.astype(jnp.bfloat16), eye_ref[...], (((0,), (0,)), ((), ())),
        preferred_element_type=jnp.float32)
    cs_row = jnp.dot(hard_row.astype(jnp.bfloat16), tri_ref[...],
                     preferred_element_type=jnp.float32)
    carry = carry_ref[0, 0]
    seg_row = carry + cs_row - hard_row
    seg_ref[...] = seg_row.astype(jnp.int32).reshape(1, 1, 1, TB)
    total = carry + jnp.sum(hard)
    carry_ref[0, 0] = total
    peritem_ref[...] = jnp.full((1, 1, 128), total, dtype=jnp.float32)


def _make_sc_pool():
    mesh = plsc.VectorSubcoreMesh(core_axis_name="c", subcore_axis_name="s")

    @functools.partial(
        pl.kernel, mesh=mesh,
        out_type=jax.ShapeDtypeStruct((B * S, D), jnp.float32),
        scratch_types=[
            pltpu.VMEM((T + 16,), jnp.int32),
            pltpu.VMEM((2, CHUNK, D), jnp.float32),
            pltpu.VMEM((GS, D), jnp.float32),
            pltpu.VMEM((GS, 16), jnp.float32),
            pltpu.SemaphoreType.DMA,
            pltpu.SemaphoreType.DMA,
        ],
    )
    def sc_pool(hid_hbm, seg_hbm, out_hbm, segs_v, rowbuf, accum, cnt,
                sem0, sem1):
        wid = lax.axis_index("s") * 2 + lax.axis_index("c")
        b = wid % B

        def sld(i):
            return segs_v[pl.ds(i, 16)][0]

        def lower_bound(target):
            def bs(_, lo_hi):
                lo, hi = lo_hi
                mid = (lo + hi) // 2
                sm = sld(mid)
                lo2 = jnp.where(sm < target, mid + 1, lo)
                hi2 = jnp.where(sm < target, hi, mid)
                return lo2, hi2
            lo, _ = lax.fori_loop(0, 11, bs, (jnp.int32(0), jnp.int32(T)))
            return lo

        pltpu.sync_copy(seg_hbm.at[pl.ds(b * T, T)], segs_v.at[pl.ds(0, T)])
        maxseg = sld(jnp.int32(T - 1))

        ones16 = jnp.ones((16,), jnp.float32)

        def zero_rows(lo, hi):
            def zero(r, _):
                for j in range(D // 16):
                    accum[r, pl.ds(16 * j, 16)] = jnp.zeros(
                        (16,), jnp.float32)
                return 0
            lax.fori_loop(lo, hi, zero, 0)

        def vtask(i, dirty):
            v = wid + NW * i
            g = v // B
            seg_base = g * GS
            used = jnp.clip(maxseg + 1 - seg_base, 0, GS)

            @pl.when(jnp.logical_and(used == 0, dirty > 0))
            def _clean():
                zero_rows(jnp.int32(0), jnp.int32(GS))

            @pl.when(used == 0)
            def _empty():
                pltpu.sync_copy(accum,
                                out_hbm.at[pl.ds(b * S + seg_base, GS)])

            @pl.when(used > 0)
            def _work():
                _run_range(seg_base, used)
            return (used > 0).astype(jnp.int32)

        def _run_range(seg_base, used):
            t0 = lower_bound(seg_base)
            t1 = lower_bound(seg_base + GS)

            a0 = (t0 // 8) * 8
            nk = (t1 - a0 + CHUNK - 1) // CHUNK

            def issue(k, slot, sem):
                ak = a0 + CHUNK * k
                ck = jnp.minimum(ak, T - CHUNK)
                pltpu.make_async_copy(
                    hid_hbm.at[pl.ds(b * T + ck, CHUNK)],
                    rowbuf.at[slot], sem).start()

            def drain(slot, sem):
                pltpu.make_async_copy(
                    hid_hbm.at[pl.ds(b * T, CHUNK)],
                    rowbuf.at[slot], sem).wait()

            @pl.when(nk > 0)
            def _prime():
                issue(0, 0, sem0)

            def chunk(k, prev):
                ak = a0 + CHUNK * k
                ck = jnp.minimum(ak, T - CHUNK)
                even = (k % 2) == 0

                @pl.when(jnp.logical_and(k + 1 < nk, even))
                def _n1():
                    issue(k + 1, 1, sem1)

                @pl.when(jnp.logical_and(k + 1 < nk,
                                         jnp.logical_not(even)))
                def _n0():
                    issue(k + 1, 0, sem0)

                @pl.when(even)
                def _w0():
                    drain(0, sem0)

                @pl.when(jnp.logical_not(even))
                def _w1():
                    drain(1, sem1)

                slot = k % 2
                begin = jnp.maximum(ak, t0)
                end = jnp.minimum(ak + CHUNK, t1)

                def tok(tt, prev):
                    sg = sld(tt)
                    r = sg - seg_base
                    row = tt - ck
                    first = sg != prev

                    @pl.when(first)
                    def _store():
                        for j in range(D // 16):
                            accum[r, pl.ds(16 * j, 16)] = (
                                rowbuf[slot, row, pl.ds(16 * j, 16)])
                        cnt[r, :] = ones16

                    @pl.when(jnp.logical_not(first))
                    def _add():
                        for j in range(D // 16):
                            plsc.addupdate(accum.at[r, pl.ds(16 * j, 16)],
                                           rowbuf[slot, row, pl.ds(16 * j, 16)])
                        plsc.addupdate(cnt.at[r], ones16)

                    return sg

                return lax.fori_loop(begin, end, tok, prev)

            lax.fori_loop(0, nk, chunk, jnp.int32(-1))

            def scale(r, _):
                inv = 1.0 / (cnt[r, :] + 1e-9)
                for j in range(D // 16):
                    accum[r, pl.ds(16 * j, 16)] = (
                        accum[r, pl.ds(16 * j, 16)] * inv)
                return 0

            lax.fori_loop(0, used, scale, 0)
            zero_rows(used, jnp.int32(GS))
            pltpu.sync_copy(accum, out_hbm.at[pl.ds(b * S + seg_base, GS)])
            return 0

        lax.fori_loop(0, (B * NG) // NW, vtask, jnp.int32(1))

    return sc_pool


_sc_pool = _make_sc_pool()


def _lgamma(x):
    z = x + 8.0
    zi = 1.0 / z
    zi2 = zi * zi
    series = ((z - 0.5) * jnp.log(z) - z + 0.9189385332046727
              + zi * (1.0 / 12.0 + zi2 * (-1.0 / 360.0 + zi2 / 1260.0)))
    prod = (x * (x + 1.0) * (x + 2.0) * (x + 3.0)
            * (x + 4.0) * (x + 5.0) * (x + 6.0) * (x + 7.0))
    return series - jnp.log(prod)


def _loss_body(nb_ref, tgt_ref, m_ref,
               loss_ref, numb_ref, totpos_ref, short_ref):
    nb = nb_ref[...]
    tgt = tgt_ref[...]
    m = m_ref[...]
    totals = jnp.sum(m, axis=1, keepdims=True)
    p = jnp.clip(tgt / totals, 1e-6, 1.0 - 1e-6)
    log_prob = (_lgamma(totals + 1.0) - _lgamma(nb + 1.0)
                - _lgamma(totals - nb + 1.0)
                + nb * jnp.log(p) + (totals - nb) * jnp.log1p(-p))
    loss = -log_prob / totals
    loss_ref[...] = jnp.full((1, 1), jnp.sum(loss) / B, dtype=jnp.float32)
    numb_ref[...] = jnp.full((1, 1), jnp.sum(nb), dtype=jnp.float32)
    totpos_ref[...] = jnp.full((1, 1), jnp.sum(m), dtype=jnp.float32)
    s_iota = jax.lax.broadcasted_iota(jnp.int32, (B, S), 1).astype(jnp.float32)
    short_ref[...] = (s_iota < nb).astype(jnp.float32)


def kernel(hidden, attention_mask, target_boundary_counts,
           W1, b1, W2, b2, gumbel_u):
    gu = gumbel_u.reshape(B, NT, TB, 1)
    mk = attention_mask.reshape(B, NT, TB, 1)
    b1r = b1.reshape(1, H)
    b2r = jnp.broadcast_to(b2.reshape(1, 1), (1, 128))

    seg_ids, peritem = pl.pallas_call(
        _seg_body,
        grid=(B, NT),
        in_specs=[
            pl.BlockSpec((1, TB, D), lambda b, t: (b, t, 0)),
            pl.BlockSpec((1, 1, TB, 1), lambda b, t: (b, t, 0, 0)),
            pl.BlockSpec((1, 1, TB, 1), lambda b, t: (b, t, 0, 0)),
            pl.BlockSpec((D, H), lambda b, t: (0, 0)),
            pl.BlockSpec((1, H), lambda b, t: (0, 0)),
            pl.BlockSpec((H, 1), lambda b, t: (0, 0)),
            pl.BlockSpec((1, 128), lambda b, t: (0, 0)),
        ],
        out_specs=[
            pl.BlockSpec((1, 1, 1, TB), lambda b, t: (b, t, 0, 0)),
            pl.BlockSpec((1, 1, 128), lambda b, t: (b, 0, 0)),
        ],
        out_shape=[
            jax.ShapeDtypeStruct((B, NT, 1, TB), jnp.int32),
            jax.ShapeDtypeStruct((B, 1, 128), jnp.float32),
        ],
        scratch_shapes=[
            pltpu.VMEM((TB, TB), jnp.bfloat16),
            pltpu.VMEM((TB, TB), jnp.bfloat16),
            pltpu.SMEM((1, 1), jnp.float32),
        ],
        compiler_params=pltpu.CompilerParams(
            dimension_semantics=("arbitrary", "arbitrary")),
    )(hidden, gu, mk, W1, b1r, W2, b2r)

    pooled = _sc_pool(hidden.reshape(B * T, D),
                      seg_ids.reshape(B * T)).reshape(B, S, D)

    nb_col = peritem[:, 0, :1]
    tgt_col = target_boundary_counts.astype(jnp.float32).reshape(B, 1)

    loss2, numb2, totpos2, shortened = pl.pallas_call(
        _loss_body,
        out_shape=[
            jax.ShapeDtypeStruct((1, 1), jnp.float32),
            jax.ShapeDtypeStruct((1, 1), jnp.float32),
            jax.ShapeDtypeStruct((1, 1), jnp.float32),
            jax.ShapeDtypeStruct((B, S), jnp.float32),
        ],
    )(nb_col, tgt_col, attention_mask)

    return (pooled, loss2[0, 0], numb2[0, 0], totpos2[0, 0], shortened)


# speedup vs baseline: 1.3192x; 1.0376x over previous
import functools

import jax
import jax.numpy as jnp
from jax import lax
from jax.experimental import pallas as pl
from jax.experimental.pallas import tpu as pltpu
from jax.experimental.pallas import tpu_sc as plsc

B, T, D, H = 8, 2048, 512, 512
S = T
TB = 1024
NT = T // TB

NW = 32
GS = 128
NG = S // GS
CHUNK = 32


def _seg_body(x_ref, u_ref, m_ref, W1_ref, b1_ref, W2_ref, b2_ref,
              seg_ref, peritem_ref, eye_ref, tri_ref, carry_ref):
    b = pl.program_id(0)
    t = pl.program_id(1)

    @pl.when(jnp.logical_and(b == 0, t == 0))
    def _build():
        ri = jax.lax.broadcasted_iota(jnp.int32, (TB, TB), 0)
        ci = jax.lax.broadcasted_iota(jnp.int32, (TB, TB), 1)
        eye_ref[...] = (ri == ci).astype(jnp.bfloat16)
        tri_ref[...] = (ri <= ci).astype(jnp.bfloat16)

    @pl.when(t == 0)
    def _reset():
        carry_ref[0, 0] = 0.0

    x = x_ref[0]
    h = jnp.maximum(
        jnp.dot(x, W1_ref[...], preferred_element_type=jnp.float32)
        + b1_ref[...], 0.0)
    logits = (jnp.dot(h, W2_ref[...], preferred_element_type=jnp.float32)
              + b2_ref[0, 0])
    u = u_ref[0, 0]
    noise = jnp.log(u) - jnp.log1p(-u)
    soft = jax.nn.sigmoid(logits + noise)
    hard = (soft > 0.5).astype(jnp.float32) * m_ref[0, 0]

    hard_row = jax.lax.dot_general(
        hard.astype(jnp.bfloat16), eye_ref[...], (((0,), (0,)), ((), ())),
        preferred_element_type=jnp.float32)
    cs_row = jnp.dot(hard_row.astype(jnp.bfloat16), tri_ref[...],
                     preferred_element_type=jnp.float32)
    carry = carry_ref[0, 0]
    seg_row = carry + cs_row - hard_row
    seg_ref[...] = seg_row.astype(jnp.int32).reshape(1, 1, 1, TB)
    total = carry + jnp.sum(hard)
    carry_ref[0, 0] = total
    peritem_ref[...] = jnp.full((1, 1, 128), total, dtype=jnp.float32)


def _make_sc_pool():
    mesh = plsc.VectorSubcoreMesh(core_axis_name="c", subcore_axis_name="s")

    @functools.partial(
        pl.kernel, mesh=mesh,
        out_type=jax.ShapeDtypeStruct((B * S, D), jnp.float32),
        scratch_types=[
            pltpu.VMEM((T + 16,), jnp.int32),
            pltpu.VMEM((2, CHUNK, D), jnp.float32),
            pltpu.VMEM((GS, D), jnp.float32),
            pltpu.VMEM((GS, 16), jnp.float32),
            pltpu.SemaphoreType.DMA,
            pltpu.SemaphoreType.DMA,
        ],
    )
    def sc_pool(hid_hbm, seg_hbm, out_hbm, segs_v, rowbuf, accum, cnt,
                sem0, sem1):
        wid = lax.axis_index("s") * 2 + lax.axis_index("c")
        b = wid % B

        def sld(i):
            return segs_v[pl.ds(i, 16)][0]

        def lower_bound(target):
            def bs(_, lo_hi):
                lo, hi = lo_hi
                mid = (lo + hi) // 2
                sm = sld(mid)
                lo2 = jnp.where(sm < target, mid + 1, lo)
                hi2 = jnp.where(sm < target, hi, mid)
                return lo2, hi2
            lo, _ = lax.fori_loop(0, 11, bs, (jnp.int32(0), jnp.int32(T)))
            return lo

        pltpu.sync_copy(seg_hbm.at[pl.ds(b * T, T)], segs_v.at[pl.ds(0, T)])
        maxseg = sld(jnp.int32(T - 1))

        ones16 = jnp.ones((16,), jnp.float32)

        def zero_rows(lo, hi):
            def zero(r, _):
                for j in range(D // 16):
                    accum[r, pl.ds(16 * j, 16)] = jnp.zeros(
                        (16,), jnp.float32)
                return 0
            lax.fori_loop(lo, hi, zero, 0)

        def vtask(i, dirty):
            v = wid + NW * i
            g = v // B
            seg_base = g * GS
            used = jnp.clip(maxseg + 1 - seg_base, 0, GS)

            @pl.when(jnp.logical_and(used == 0, dirty > 0))
            def _clean():
                zero_rows(jnp.int32(0), jnp.int32(GS))

            @pl.when(used == 0)
            def _empty():
                pltpu.sync_copy(accum,
                                out_hbm.at[pl.ds(b * S + seg_base, GS)])

            @pl.when(used > 0)
            def _work():
                _run_range(seg_base, used)
            return (used > 0).astype(jnp.int32)

        def _run_range(seg_base, used):
            t0 = lower_bound(seg_base)
            t1 = lower_bound(seg_base + GS)

            a0 = (t0 // 8) * 8
            nk = (t1 - a0 + CHUNK - 1) // CHUNK

            def issue(k, slot, sem):
                ak = a0 + CHUNK * k
                ck = jnp.minimum(ak, T - CHUNK)
                pltpu.make_async_copy(
                    hid_hbm.at[pl.ds(b * T + ck, CHUNK)],
                    rowbuf.at[slot], sem).start()

            def drain(slot, sem):
                pltpu.make_async_copy(
                    hid_hbm.at[pl.ds(b * T, CHUNK)],
                    rowbuf.at[slot], sem).wait()

            @pl.when(nk > 0)
            def _prime():
                issue(0, 0, sem0)

            def chunk(k, prev):
                ak = a0 + CHUNK * k
                ck = jnp.minimum(ak, T - CHUNK)
                even = (k % 2) == 0

                @pl.when(jnp.logical_and(k + 1 < nk, even))
                def _n1():
                    issue(k + 1, 1, sem1)

                @pl.when(jnp.logical_and(k + 1 < nk,
                                         jnp.logical_not(even)))
                def _n0():
                    issue(k + 1, 0, sem0)

                @pl.when(even)
                def _w0():
                    drain(0, sem0)

                @pl.when(jnp.logical_not(even))
                def _w1():
                    drain(1, sem1)

                slot = k % 2
                begin = jnp.maximum(ak, t0)
                end = jnp.minimum(ak + CHUNK, t1)

                def tok(tt, prev):
                    sg = sld(tt)
                    r = sg - seg_base
                    row = tt - ck
                    first = sg != prev

                    @pl.when(first)
                    def _store():
                        for j in range(D // 16):
                            accum[r, pl.ds(16 * j, 16)] = (
                                rowbuf[slot, row, pl.ds(16 * j, 16)])
                        cnt[r, :] = ones16

                    @pl.when(jnp.logical_not(first))
                    def _add():
                        for j in range(D // 16):
                            plsc.addupdate(accum.at[r, pl.ds(16 * j, 16)],
                                           rowbuf[slot, row, pl.ds(16 * j, 16)])
                        plsc.addupdate(cnt.at[r], ones16)

                    return sg

                return lax.fori_loop(begin, end, tok, prev)

            lax.fori_loop(0, nk, chunk, jnp.int32(-1))

            def scale(r, _):
                inv = 1.0 / (cnt[r, :] + 1e-9)
                for j in range(D // 16):
                    accum[r, pl.ds(16 * j, 16)] = (
                        accum[r, pl.ds(16 * j, 16)] * inv)
                return 0

            lax.fori_loop(0, used, scale, 0)
            zero_rows(used, jnp.int32(GS))
            pltpu.sync_copy(accum, out_hbm.at[pl.ds(b * S + seg_base, GS)])
            return 0

        lax.fori_loop(0, (B * NG) // NW, vtask, jnp.int32(1))

    return sc_pool


_sc_pool = _make_sc_pool()


def _lgamma(x):
    z = x + 8.0
    zi = 1.0 / z
    zi2 = zi * zi
    series = ((z - 0.5) * jnp.log(z) - z + 0.9189385332046727
              + zi * (1.0 / 12.0 + zi2 * (-1.0 / 360.0 + zi2 / 1260.0)))
    prod = (x * (x + 1.0) * (x + 2.0) * (x + 3.0)
            * (x + 4.0) * (x + 5.0) * (x + 6.0) * (x + 7.0))
    return series - jnp.log(prod)


def _loss_body(nb_ref, tgt_ref, m_ref,
               loss_ref, numb_ref, totpos_ref, short_ref):
    nb = nb_ref[...]
    tgt = tgt_ref[...]
    m = m_ref[...]
    totals = jnp.sum(m, axis=1, keepdims=True)
    p = jnp.clip(tgt / totals, 1e-6, 1.0 - 1e-6)
    log_prob = (_lgamma(totals + 1.0) - _lgamma(nb + 1.0)
                - _lgamma(totals - nb + 1.0)
                + nb * jnp.log(p) + (totals - nb) * jnp.log1p(-p))
    loss = -log_prob / totals
    loss_ref[...] = jnp.full((1, 1), jnp.sum(loss) / B, dtype=jnp.float32)
    numb_ref[...] = jnp.full((1, 1), jnp.sum(nb), dtype=jnp.float32)
    totpos_ref[...] = jnp.full((1, 1), jnp.sum(m), dtype=jnp.float32)
    s_iota = jax.lax.broadcasted_iota(jnp.int32, (B, S), 1).astype(jnp.float32)
    short_ref[...] = (s_iota < nb).astype(jnp.float32)


def kernel(hidden, attention_mask, target_boundary_counts,
           W1, b1, W2, b2, gumbel_u):
    gu = gumbel_u.reshape(B, NT, TB, 1)
    mk = attention_mask.reshape(B, NT, TB, 1)
    b1r = b1.reshape(1, H)
    b2r = jnp.broadcast_to(b2.reshape(1, 1), (1, 128))

    seg_ids, peritem = pl.pallas_call(
        _seg_body,
        grid=(B, NT),
        in_specs=[
            pl.BlockSpec((1, TB, D), lambda b, t: (b, t, 0)),
            pl.BlockSpec((1, 1, TB, 1), lambda b, t: (b, t, 0, 0)),
            pl.BlockSpec((1, 1, TB, 1), lambda b, t: (b, t, 0, 0)),
            pl.BlockSpec((D, H), lambda b, t: (0, 0)),
            pl.BlockSpec((1, H), lambda b, t: (0, 0)),
            pl.BlockSpec((H, 1), lambda b, t: (0, 0)),
            pl.BlockSpec((1, 128), lambda b, t: (0, 0)),
        ],
        out_specs=[
            pl.BlockSpec((1, 1, 1, TB), lambda b, t: (b, t, 0, 0)),
            pl.BlockSpec((1, 1, 128), lambda b, t: (b, 0, 0)),
        ],
        out_shape=[
            jax.ShapeDtypeStruct((B, NT, 1, TB), jnp.int32),
            jax.ShapeDtypeStruct((B, 1, 128), jnp.float32),
        ],
        scratch_shapes=[
            pltpu.VMEM((TB, TB), jnp.bfloat16),
            pltpu.VMEM((TB, TB), jnp.bfloat16),
            pltpu.SMEM((1, 1), jnp.float32),
        ],
        compiler_params=pltpu.CompilerParams(
            dimension_semantics=("arbitrary", "arbitrary")),
    )(hidden, gu, mk, W1, b1r, W2, b2r)

    pooled = _sc_pool(hidden.reshape(B * T, D),
                      seg_ids.reshape(B * T)).reshape(B, S, D)

    nb_col = peritem[:, 0, :1]
    tgt_col = target_boundary_counts.astype(jnp.float32).reshape(B, 1)

    loss2, numb2, totpos2, shortened = pl.pallas_call(
        _loss_body,
        out_shape=[
            jax.ShapeDtypeStruct((1, 1), jnp.float32),
            jax.ShapeDtypeStruct((1, 1), jnp.float32),
            jax.ShapeDtypeStruct((1, 1), jnp.float32),
            jax.ShapeDtypeStruct((B, S), jnp.float32),
        ],
    )(nb_col, tgt_col, attention_mask)

    return (pooled, loss2[0, 0], numb2[0, 0], totpos2[0, 0], shortened)
